# baseline (speedup 1.0000x reference)
import jax
import jax.numpy as jnp
from jax.experimental import pallas as pl

_BLOCK_ROWS = 2048


def _copy_body(t_ref, o_ref):
    o_ref[...] = t_ref[...]


def kernel(x, table):
    seq_len = x.shape[1]
    dim = table.shape[1]
    return pl.pallas_call(
        _copy_body,
        grid=(seq_len // _BLOCK_ROWS,),
        in_specs=[pl.BlockSpec((_BLOCK_ROWS, dim), lambda i: (i, 0))],
        out_specs=pl.BlockSpec((_BLOCK_ROWS, dim), lambda i: (i, 0)),
        out_shape=jax.ShapeDtypeStruct((seq_len, dim), table.dtype),
    )(table)
